# baseline (speedup 1.0000x reference)
import functools

import jax
import jax.numpy as jnp
from jax import lax
from jax.experimental import pallas as pl
from jax.experimental.pallas import tpu as pltpu
from jax.experimental.pallas import tpu_sc as plsc

B = 4
T = 8192
D = 768
ROWS = B * T
NW = 32
TPW = T // NW
W = 8
CR = B * W
NCHUNK = TPW // W
NBUF = 4
K = NBUF - 1


def _issue_gather(tok_hbm, idx_v, rows_v, buf, c, gsems):
    pltpu.async_copy(
        tok_hbm.at[idx_v.at[pl.ds(c * CR, CR)]],
        rows_v.at[buf],
        gsems[buf],
    )


def _wait_gather(tok_hbm, idx_v, rows_v, buf, c, gsems):
    pltpu.make_async_copy(
        tok_hbm.at[idx_v.at[pl.ds(c * CR, CR)]],
        rows_v.at[buf],
        gsems[buf],
    ).wait()


def _issue_pos(pos_hbm, pos_v, buf, t0, c, psems):
    pltpu.async_copy(pos_hbm.at[pl.ds(t0 + c * W, W)], pos_v.at[buf],
                     psems[buf])


def _wait_pos(pos_hbm, pos_v, buf, t0, c, psems):
    pltpu.make_async_copy(
        pos_hbm.at[pl.ds(t0 + c * W, W)], pos_v.at[buf], psems[buf]
    ).wait()


def _issue_stores(out_hbm, rows_v, buf, t0, c, ssems):
    for b in range(B):
        pltpu.async_copy(
            rows_v.at[buf, pl.ds(b * W, W)],
            out_hbm.at[pl.ds(b * T + t0 + c * W, W)],
            ssems[buf],
        )


def _wait_stores(out_hbm, rows_v, buf, t0, c, ssems):
    for b in range(B):
        pltpu.make_async_copy(
            rows_v.at[buf, pl.ds(b * W, W)],
            out_hbm.at[pl.ds(b * T + t0 + c * W, W)],
            ssems[buf],
        ).wait()


def _compute(rows_v, pos_v, buf):
    @pl.loop(0, W)
    def _row(r):
        for d in range(0, D, 16):
            sl = pl.ds(d, 16)
            pv = pos_v[buf, r, sl]
            for b in range(B):
                plsc.addupdate(rows_v.at[buf, b * W + r, sl], pv)


def _emb_kernel(tok_hbm, idx_hbm, pos_hbm, out_hbm, idx_v, rows_v, pos_v,
                gsem0, gsem1, gsem2, gsem3, psem0, psem1, psem2, psem3,
                ssem0, ssem1, ssem2, ssem3):
    wid = lax.axis_index("s") * 2 + lax.axis_index("c")
    t0 = wid * TPW
    gsems = (gsem0, gsem1, gsem2, gsem3)
    psems = (psem0, psem1, psem2, psem3)
    ssems = (ssem0, ssem1, ssem2, ssem3)

    pltpu.sync_copy(idx_hbm.at[pl.ds(wid * B * TPW, B * TPW)], idx_v)

    def head(c):
        nb = (c + K) % NBUF
        if c >= 1:
            _wait_stores(out_hbm, rows_v, nb, t0, c - 1, ssems)
        _issue_gather(tok_hbm, idx_v, rows_v, nb, c + K, gsems)
        _issue_pos(pos_hbm, pos_v, nb, t0, c + K, psems)

    def tail(c):
        P = c % NBUF
        _wait_gather(tok_hbm, idx_v, rows_v, P, c, gsems)
        _wait_pos(pos_hbm, pos_v, P, t0, c, psems)
        _compute(rows_v, pos_v, P)
        _issue_stores(out_hbm, rows_v, P, t0, c, ssems)

    for c in range(K):
        _issue_gather(tok_hbm, idx_v, rows_v, c, c, gsems)
        _issue_pos(pos_hbm, pos_v, c, t0, c, psems)
    for c in range(NBUF):
        head(c)
        tail(c)

    @pl.loop(NBUF, NCHUNK - K - NBUF + 1, step=NBUF)
    def _chunks(c0):
        for cp in range(NBUF):
            c = c0 + cp
            nb = (cp + K) % NBUF
            _wait_stores(out_hbm, rows_v, nb, t0, c - 1, ssems)
            _issue_gather(tok_hbm, idx_v, rows_v, nb, c + K, gsems)
            _issue_pos(pos_hbm, pos_v, nb, t0, c + K, psems)
            P = cp
            _wait_gather(tok_hbm, idx_v, rows_v, P, c, gsems)
            _wait_pos(pos_hbm, pos_v, P, t0, c, psems)
            _compute(rows_v, pos_v, P)
            _issue_stores(out_hbm, rows_v, P, t0, c, ssems)

    head(NCHUNK - NBUF)
    for c in range(NCHUNK - NBUF, NCHUNK):
        tail(c)
    for c in range(NCHUNK - NBUF, NCHUNK):
        _wait_stores(out_hbm, rows_v, c % NBUF, t0, c, ssems)


@jax.jit
def kernel(x, token_emb, pos_emb):
    xs = (x.astype(jnp.int32)
          .reshape(B, NW, NCHUNK, W)
          .transpose(1, 2, 0, 3)
          .reshape(ROWS))
    mesh = plsc.VectorSubcoreMesh(core_axis_name="c", subcore_axis_name="s")
    run = functools.partial(
        pl.kernel,
        out_type=jax.ShapeDtypeStruct((ROWS, D), jnp.float32),
        mesh=mesh,
        scratch_types=[
            pltpu.VMEM((B * TPW,), jnp.int32),
            pltpu.VMEM((NBUF, CR, D), jnp.float32),
            pltpu.VMEM((NBUF, W, D), jnp.float32),
        ] + [pltpu.SemaphoreType.DMA] * (3 * NBUF),
    )(_emb_kernel)
    out = run(token_emb, xs, pos_emb)
    return out.reshape(B, T, D)


# speedup vs baseline: 1.0427x; 1.0427x over previous
import functools

import jax
import jax.numpy as jnp
from jax import lax
from jax.experimental import pallas as pl
from jax.experimental.pallas import tpu as pltpu
from jax.experimental.pallas import tpu_sc as plsc

B = 4
T = 8192
D = 768
ROWS = B * T
NW = 32
TPW = T // NW
W = 8
CR = B * W
NCHUNK = TPW // W
NBUF = 3


def _issue_gather(tok_hbm, idx_v, rows_v, buf, c, gsems):
    pltpu.async_copy(
        tok_hbm.at[idx_v.at[pl.ds(c * CR, CR)]],
        rows_v.at[buf],
        gsems[buf],
    )


def _wait_gather(tok_hbm, idx_v, rows_v, buf, c, gsems):
    pltpu.make_async_copy(
        tok_hbm.at[idx_v.at[pl.ds(c * CR, CR)]],
        rows_v.at[buf],
        gsems[buf],
    ).wait()


def _issue_pos(pos_hbm, pos_v, buf, t0, c, psems):
    pltpu.async_copy(pos_hbm.at[pl.ds(t0 + c * W, W)], pos_v.at[buf],
                     psems[buf])


def _wait_pos(pos_hbm, pos_v, buf, t0, c, psems):
    pltpu.make_async_copy(
        pos_hbm.at[pl.ds(t0 + c * W, W)], pos_v.at[buf], psems[buf]
    ).wait()


def _issue_stores(out_hbm, rows_v, buf, t0, c, ssems):
    for b in range(B):
        pltpu.async_copy(
            rows_v.at[buf, pl.ds(b * W, W)],
            out_hbm.at[pl.ds(b * T + t0 + c * W, W)],
            ssems[buf],
        )


def _wait_stores(out_hbm, rows_v, buf, t0, c, ssems):
    for b in range(B):
        pltpu.make_async_copy(
            rows_v.at[buf, pl.ds(b * W, W)],
            out_hbm.at[pl.ds(b * T + t0 + c * W, W)],
            ssems[buf],
        ).wait()


def _compute(rows_v, pos_v, buf):
    @pl.loop(0, D, step=16)
    def _dslice(d):
        sl = pl.ds(d, 16)
        for r in range(W):
            pv = pos_v[buf, r, sl]
            for b in range(B):
                plsc.addupdate(rows_v.at[buf, b * W + r, sl], pv)


def _emb_kernel(tok_hbm, idx_hbm, pos_hbm, out_hbm, idx_v, rows_v, pos_v,
                gsem0, gsem1, gsem2, psem0, psem1, psem2,
                ssem0, ssem1, ssem2):
    wid = lax.axis_index("s") * 2 + lax.axis_index("c")
    t0 = wid * TPW
    gsems = (gsem0, gsem1, gsem2)
    psems = (psem0, psem1, psem2)
    ssems = (ssem0, ssem1, ssem2)

    pltpu.sync_copy(idx_hbm.at[pl.ds(wid * B * TPW, B * TPW)], idx_v)

    def head(c):
        nb = (c + 2) % NBUF
        if c >= 1:
            _wait_stores(out_hbm, rows_v, nb, t0, c - 1, ssems)
        _issue_gather(tok_hbm, idx_v, rows_v, nb, c + 2, gsems)
        _issue_pos(pos_hbm, pos_v, nb, t0, c + 2, psems)

    def tail(c):
        P = c % NBUF
        _wait_gather(tok_hbm, idx_v, rows_v, P, c, gsems)
        _wait_pos(pos_hbm, pos_v, P, t0, c, psems)
        _compute(rows_v, pos_v, P)
        _issue_stores(out_hbm, rows_v, P, t0, c, ssems)

    for c in range(2):
        _issue_gather(tok_hbm, idx_v, rows_v, c, c, gsems)
        _issue_pos(pos_hbm, pos_v, c, t0, c, psems)
    for c in range(NBUF):
        head(c)
        tail(c)

    @pl.loop(NBUF, NCHUNK - 2, step=NBUF)
    def _chunks(c0):
        for cp in range(NBUF):
            c = c0 + cp
            nb = (cp + 2) % NBUF
            _wait_stores(out_hbm, rows_v, nb, t0, c - 1, ssems)
            _issue_gather(tok_hbm, idx_v, rows_v, nb, c + 2, gsems)
            _issue_pos(pos_hbm, pos_v, nb, t0, c + 2, psems)
            P = cp
            _wait_gather(tok_hbm, idx_v, rows_v, P, c, gsems)
            _wait_pos(pos_hbm, pos_v, P, t0, c, psems)
            _compute(rows_v, pos_v, P)
            _issue_stores(out_hbm, rows_v, P, t0, c, ssems)

    for c in (NCHUNK - 2, NCHUNK - 1):
        tail(c)
    for c in (NCHUNK - 3, NCHUNK - 2, NCHUNK - 1):
        _wait_stores(out_hbm, rows_v, c % NBUF, t0, c, ssems)


@jax.jit
def kernel(x, token_emb, pos_emb):
    xs = (x.astype(jnp.int32)
          .reshape(B, NW, NCHUNK, W)
          .transpose(1, 2, 0, 3)
          .reshape(ROWS))
    mesh = plsc.VectorSubcoreMesh(core_axis_name="c", subcore_axis_name="s")
    run = functools.partial(
        pl.kernel,
        out_type=jax.ShapeDtypeStruct((ROWS, D), jnp.float32),
        mesh=mesh,
        scratch_types=[
            pltpu.VMEM((B * TPW,), jnp.int32),
            pltpu.VMEM((NBUF, CR, D), jnp.float32),
            pltpu.VMEM((NBUF, W, D), jnp.float32),
            pltpu.SemaphoreType.DMA,
            pltpu.SemaphoreType.DMA,
            pltpu.SemaphoreType.DMA,
            pltpu.SemaphoreType.DMA,
            pltpu.SemaphoreType.DMA,
            pltpu.SemaphoreType.DMA,
            pltpu.SemaphoreType.DMA,
            pltpu.SemaphoreType.DMA,
            pltpu.SemaphoreType.DMA,
        ],
    )(_emb_kernel)
    out = run(token_emb, xs, pos_emb)
    return out.reshape(B, T, D)
